# baseline (speedup 1.0000x reference)
import jax
import jax.numpy as jnp
from jax.experimental import pallas as pl

M, B, D = 65536, 8192, 256
BLK = 2048
NM = M // BLK
NB = B // BLK


def _copy_body(k_ref, v_ref, nk_ref, nv_ref, ok_ref, ov_ref):
    i = pl.program_id(0)

    @pl.when(i < NM)
    def _():
        ok_ref[...] = k_ref[...]
        ov_ref[...] = v_ref[...]

    @pl.when(i >= NM)
    def _():
        ok_ref[...] = nk_ref[...]
        ov_ref[...] = nv_ref[...]


def kernel(keys, vals, new_keys, new_vals):
    grid = (NM + NB,)
    old_spec = pl.BlockSpec((BLK, D), lambda i: (jnp.minimum(i, NM - 1), 0))
    new_spec = pl.BlockSpec((BLK, D), lambda i: (jnp.maximum(i - NM, 0), 0))
    out_spec = pl.BlockSpec((BLK, D), lambda i: (i, 0))
    out_shape = jax.ShapeDtypeStruct((M + B, D), keys.dtype)
    return pl.pallas_call(
        _copy_body,
        grid=grid,
        in_specs=[old_spec, old_spec, new_spec, new_spec],
        out_specs=[out_spec, out_spec],
        out_shape=[out_shape, out_shape],
    )(keys, vals, new_keys, new_vals)
